# Initial kernel scaffold; baseline (speedup 1.0000x reference)
import jax
import jax.numpy as jnp
from jax.experimental import pallas as pl


def kernel(initial_e_emb, edge_index, W1, b1, W2, b2, Wu, bu, Ws, bs):
    raise NotImplementedError("write your pallas kernel here")


# speedup vs baseline: 14.8612x; 14.8612x over previous
import functools

import jax
import jax.numpy as jnp
from jax import lax
from jax.experimental import pallas as pl
from jax.experimental.pallas import tpu as pltpu
from jax.experimental.pallas import tpu_sc as plsc

_N = 10000
_D = 128
_E = 320000
_NC = 2
_NS = 16
_NW = _NC * _NS
_CHUNK = 128
_NCHUNKS = _E // _CHUNK
_CBASE = _NCHUNKS // _NW
_CREM = _NCHUNKS % _NW
_RPS = 624
_TAIL0 = _RPS * _NS
_TAILN = _N - _TAIL0
_R = 1000

_mesh = plsc.VectorSubcoreMesh(core_axis_name="c", subcore_axis_name="s")


@functools.partial(
    pl.kernel,
    out_type=jax.ShapeDtypeStruct((_NC, _N, _D), jnp.float32),
    mesh=_mesh,
    scratch_types=[
        pltpu.VMEM((1, _CHUNK), jnp.int32),
        pltpu.VMEM((_CHUNK, _D), jnp.float32),
        pltpu.VMEM_SHARED((_N, _D), jnp.float32),
    ],
)
def _sc_degree(dst_hbm, ones_hbm, z_hbm, out_hbm, dbuf, ones_v, acc):
    cid = lax.axis_index("c")
    sid = lax.axis_index("s")
    wid = cid * _NS + sid
    r0 = sid * _RPS
    pltpu.sync_copy(z_hbm.at[pl.ds(r0, _RPS)], acc.at[pl.ds(r0, _RPS)])

    @pl.when(sid == _NS - 1)
    def _():
        pltpu.sync_copy(z_hbm.at[pl.ds(_TAIL0, _TAILN)],
                        acc.at[pl.ds(_TAIL0, _TAILN)])

    pltpu.sync_copy(ones_hbm, ones_v)
    plsc.subcore_barrier()
    nch = _CBASE + jnp.where(wid < _CREM, 1, 0)
    start = wid * _CBASE + jnp.minimum(wid, _CREM)

    @pl.loop(0, _CBASE + 1)
    def _(i):
        @pl.when(i < nch)
        def _():
            off = (start + i) * _CHUNK
            pltpu.sync_copy(dst_hbm.at[pl.ds(off, _CHUNK)], dbuf.at[0])
            pltpu.sync_copy(ones_v, acc.at[dbuf.at[0]], add=True)

    plsc.subcore_barrier()
    pltpu.sync_copy(acc.at[pl.ds(r0, _RPS)], out_hbm.at[cid, pl.ds(r0, _RPS)])

    @pl.when(sid == _NS - 1)
    def _():
        pltpu.sync_copy(acc.at[pl.ds(_TAIL0, _TAILN)],
                        out_hbm.at[cid, pl.ds(_TAIL0, _TAILN)])


@functools.partial(
    pl.kernel,
    out_type=jax.ShapeDtypeStruct((_NC, _N, _D), jnp.float32),
    mesh=_mesh,
    scratch_types=[
        pltpu.VMEM((1, _CHUNK), jnp.int32),
        pltpu.VMEM((1, _CHUNK), jnp.int32),
        pltpu.VMEM((1, _CHUNK, _D), jnp.float32),
        pltpu.VMEM_SHARED((_N, _D), jnp.float32),
        pltpu.SemaphoreType.DMA,
    ],
)
def _sc_agg(hs_hbm, src_hbm, dst_hbm, z_hbm, out_hbm, sbuf, dbuf, rbuf, acc, sem):
    cid = lax.axis_index("c")
    sid = lax.axis_index("s")
    wid = cid * _NS + sid
    r0 = sid * _RPS
    pltpu.sync_copy(z_hbm.at[pl.ds(r0, _RPS)], acc.at[pl.ds(r0, _RPS)])

    @pl.when(sid == _NS - 1)
    def _():
        pltpu.sync_copy(z_hbm.at[pl.ds(_TAIL0, _TAILN)],
                        acc.at[pl.ds(_TAIL0, _TAILN)])

    plsc.subcore_barrier()
    nch = _CBASE + jnp.where(wid < _CREM, 1, 0)
    start = wid * _CBASE + jnp.minimum(wid, _CREM)

    @pl.loop(0, _CBASE + 1)
    def _(i):
        @pl.when(i < nch)
        def _():
            off = (start + i) * _CHUNK
            pltpu.sync_copy(src_hbm.at[pl.ds(off, _CHUNK)], sbuf.at[0])
            pltpu.sync_copy(dst_hbm.at[pl.ds(off, _CHUNK)], dbuf.at[0])
            pltpu.async_copy(hs_hbm.at[sbuf.at[0]], rbuf.at[0], sem).wait()
            pltpu.sync_copy(rbuf.at[0], acc.at[dbuf.at[0]], add=True)

    plsc.subcore_barrier()
    pltpu.sync_copy(acc.at[pl.ds(r0, _RPS)], out_hbm.at[cid, pl.ds(r0, _RPS)])

    @pl.when(sid == _NS - 1)
    def _():
        pltpu.sync_copy(acc.at[pl.ds(_TAIL0, _TAILN)],
                        out_hbm.at[cid, pl.ds(_TAIL0, _TAILN)])


def _dis_from(d_ref):
    deg = 1.0 + d_ref[0, :, 0:1] + d_ref[1, :, 0:1]
    return lax.rsqrt(deg)


def _k1_body(x_ref, w_ref, d_ref, o_ref):
    dis = _dis_from(d_ref)
    o_ref[...] = dis * jnp.dot(x_ref[...], w_ref[...],
                               preferred_element_type=jnp.float32)


def _k2_body(p_ref, hs_ref, d_ref, b1_ref, w2_ref, o_ref):
    dis = _dis_from(d_ref)
    s = p_ref[0] + p_ref[1] + hs_ref[...]
    h = jnp.maximum(dis * s + b1_ref[...], 0.0)
    o_ref[...] = dis * jnp.dot(h, w2_ref[...],
                               preferred_element_type=jnp.float32)


def _k3_body(q_ref, hs_ref, d_ref, b2_ref, wu_ref, bu_ref, ws_ref, bs_ref,
             u_ref, s_ref):
    dis = _dis_from(d_ref)
    hh = dis * (q_ref[0] + q_ref[1] + hs_ref[...]) + b2_ref[...]
    u_ref[...] = jnp.dot(hh, wu_ref[...],
                         preferred_element_type=jnp.float32) + bu_ref[...]
    sg = jnp.dot(hh, ws_ref[...],
                 preferred_element_type=jnp.float32) + bs_ref[...]
    s_ref[...] = jnp.maximum(sg, 0.01)


_row = pl.BlockSpec((_R, _D), lambda i: (i, 0))
_full = pl.BlockSpec((_D, _D), lambda i: (0, 0))
_bias = pl.BlockSpec((1, _D), lambda i: (0, 0))
_part = pl.BlockSpec((_NC, _R, _D), lambda i: (0, i, 0))
_degs = _part
_out = jax.ShapeDtypeStruct((_N, _D), jnp.float32)


def _k1(x, W1, degs):
    return pl.pallas_call(
        _k1_body, grid=(_N // _R,),
        in_specs=[_row, _full, _degs],
        out_specs=_row, out_shape=_out,
    )(x, W1, degs)


def _k2(p, hs1, degs, b1, W2):
    return pl.pallas_call(
        _k2_body, grid=(_N // _R,),
        in_specs=[_part, _row, _degs, _bias, _full],
        out_specs=_row, out_shape=_out,
    )(p, hs1, degs, b1, W2)


def _k3(q, hs2, degs, b2, Wu, bu, Ws, bs):
    return pl.pallas_call(
        _k3_body, grid=(_N // _R,),
        in_specs=[_part, _row, _degs, _bias, _full, _bias, _full, _bias],
        out_specs=(_row, _row), out_shape=(_out, _out),
    )(q, hs2, degs, b2, Wu, bu, Ws, bs)


def kernel(initial_e_emb, edge_index, W1, b1, W2, b2, Wu, bu, Ws, bs):
    ei = edge_index.astype(jnp.int32)
    src, dst = ei[0], ei[1]
    zeros = jnp.zeros((_N, _D), jnp.float32)
    ones = jnp.ones((_CHUNK, _D), jnp.float32)

    degs = _sc_degree(dst, ones, zeros)
    hs1 = _k1(initial_e_emb, W1, degs)
    p = _sc_agg(hs1, src, dst, zeros)
    hs2 = _k2(p, hs1, degs, b1.reshape(1, _D), W2)
    q = _sc_agg(hs2, src, dst, zeros)
    u, sigma = _k3(q, hs2, degs, b2.reshape(1, _D),
                   Wu, bu.reshape(1, _D), Ws, bs.reshape(1, _D))
    return (u, sigma)


# speedup vs baseline: 26.7798x; 1.8020x over previous
import functools

import jax
import jax.numpy as jnp
from jax import lax
from jax.experimental import pallas as pl
from jax.experimental.pallas import tpu as pltpu
from jax.experimental.pallas import tpu_sc as plsc

_N = 10000
_D = 128
_E = 320000
_NC = 2
_NS = 16
_NW = _NC * _NS
_CHUNK = 128
_WCH = 80
_NCH = _E // _CHUNK
_EP = _NW * _WCH * _CHUNK
_RPS = 624
_TAIL0 = _RPS * _NS
_TAILN = _N - _TAIL0
_R = 1000

_mesh = plsc.VectorSubcoreMesh(core_axis_name="c", subcore_axis_name="s")


def _zero_acc(z_hbm, acc, sid):
    r0 = sid * _RPS
    pltpu.sync_copy(z_hbm.at[pl.ds(r0, _RPS)], acc.at[pl.ds(r0, _RPS)])

    @pl.when(sid == _NS - 1)
    def _():
        pltpu.sync_copy(z_hbm.at[pl.ds(_TAIL0, _TAILN)],
                        acc.at[pl.ds(_TAIL0, _TAILN)])


def _writeback(acc, out_hbm, cid, sid):
    r0 = sid * _RPS
    pltpu.sync_copy(acc.at[pl.ds(r0, _RPS)], out_hbm.at[cid, pl.ds(r0, _RPS)])

    @pl.when(sid == _NS - 1)
    def _():
        pltpu.sync_copy(acc.at[pl.ds(_TAIL0, _TAILN)],
                        out_hbm.at[cid, pl.ds(_TAIL0, _TAILN)])


@functools.partial(
    pl.kernel,
    out_type=jax.ShapeDtypeStruct((_NC, _N, _D), jnp.float32),
    mesh=_mesh,
    scratch_types=[
        pltpu.VMEM((_WCH, _CHUNK), jnp.int32),
        pltpu.VMEM((_CHUNK, _D), jnp.float32),
        pltpu.VMEM_SHARED((_N, _D), jnp.float32),
    ],
)
def _sc_degree(dst_hbm, ones_hbm, z_hbm, out_hbm, dbuf, ones_v, acc):
    cid = lax.axis_index("c")
    sid = lax.axis_index("s")
    wid = cid * _NS + sid
    nch = jnp.minimum(_WCH, _NCH - wid * _WCH)
    pltpu.sync_copy(dst_hbm.at[pl.ds(wid * _WCH, _WCH)], dbuf)
    _zero_acc(z_hbm, acc, sid)
    pltpu.sync_copy(ones_hbm, ones_v)
    plsc.subcore_barrier()

    @pl.loop(0, _WCH)
    def _(j):
        @pl.when(j < nch)
        def _():
            pltpu.sync_copy(ones_v, acc.at[dbuf.at[j]], add=True)

    plsc.subcore_barrier()
    _writeback(acc, out_hbm, cid, sid)


@functools.partial(
    pl.kernel,
    out_type=jax.ShapeDtypeStruct((_NC, _N, _D), jnp.float32),
    mesh=_mesh,
    scratch_types=[
        pltpu.VMEM((_WCH // 2, _CHUNK), jnp.int32),
        pltpu.VMEM((_WCH // 2, _CHUNK), jnp.int32),
        pltpu.VMEM((2, _CHUNK, _D), jnp.float32),
        pltpu.VMEM_SHARED((_N, _D), jnp.float32),
        pltpu.SemaphoreType.DMA,
        pltpu.SemaphoreType.DMA,
    ],
)
def _sc_agg(hs_hbm, src_hbm, dst_hbm, z_hbm, out_hbm, sbuf, dbuf, rbuf, acc,
            sem0, sem1):
    cid = lax.axis_index("c")
    sid = lax.axis_index("s")
    wid = cid * _NS + sid
    nch = jnp.minimum(_WCH, _NCH - wid * _WCH)
    _zero_acc(z_hbm, acc, sid)
    plsc.subcore_barrier()

    hw = _WCH // 2
    for p in range(2):
        cnt = jnp.clip(nch - p * hw, 0, hw)

        @pl.when(cnt > 0)
        def _():
            base = wid * _WCH + p * hw
            pltpu.sync_copy(src_hbm.at[pl.ds(base, hw)], sbuf)
            pltpu.sync_copy(dst_hbm.at[pl.ds(base, hw)], dbuf)
            pltpu.async_copy(hs_hbm.at[sbuf.at[0]], rbuf.at[0], sem0)

            @pl.loop(0, hw // 2)
            def _(k):
                @pl.when(k < cnt // 2)
                def _():
                    j = 2 * k
                    c1 = pltpu.async_copy(hs_hbm.at[sbuf.at[j + 1]],
                                          rbuf.at[1], sem1)
                    pltpu.make_async_copy(hs_hbm.at[sbuf.at[j]], rbuf.at[0],
                                          sem0).wait()
                    pltpu.sync_copy(rbuf.at[0], acc.at[dbuf.at[j]], add=True)

                    @pl.when(j + 2 < cnt)
                    def _():
                        pltpu.async_copy(hs_hbm.at[sbuf.at[j + 2]],
                                         rbuf.at[0], sem0)

                    c1.wait()
                    pltpu.sync_copy(rbuf.at[1], acc.at[dbuf.at[j + 1]],
                                    add=True)

    plsc.subcore_barrier()
    _writeback(acc, out_hbm, cid, sid)


def _dis_from(d_ref):
    deg = 1.0 + d_ref[0, :, 0:1] + d_ref[1, :, 0:1]
    return lax.rsqrt(deg)


def _k1_body(x_ref, w_ref, d_ref, o_ref):
    dis = _dis_from(d_ref)
    o_ref[...] = dis * jnp.dot(x_ref[...], w_ref[...],
                               preferred_element_type=jnp.float32)


def _k2_body(p_ref, hs_ref, d_ref, b1_ref, w2_ref, o_ref):
    dis = _dis_from(d_ref)
    s = p_ref[0] + p_ref[1] + hs_ref[...]
    h = jnp.maximum(dis * s + b1_ref[...], 0.0)
    o_ref[...] = dis * jnp.dot(h, w2_ref[...],
                               preferred_element_type=jnp.float32)


def _k3_body(q_ref, hs_ref, d_ref, b2_ref, wu_ref, bu_ref, ws_ref, bs_ref,
             u_ref, s_ref):
    dis = _dis_from(d_ref)
    hh = dis * (q_ref[0] + q_ref[1] + hs_ref[...]) + b2_ref[...]
    u_ref[...] = jnp.dot(hh, wu_ref[...],
                         preferred_element_type=jnp.float32) + bu_ref[...]
    sg = jnp.dot(hh, ws_ref[...],
                 preferred_element_type=jnp.float32) + bs_ref[...]
    s_ref[...] = jnp.maximum(sg, 0.01)


_row = pl.BlockSpec((_R, _D), lambda i: (i, 0))
_full = pl.BlockSpec((_D, _D), lambda i: (0, 0))
_bias = pl.BlockSpec((1, _D), lambda i: (0, 0))
_part = pl.BlockSpec((_NC, _R, _D), lambda i: (0, i, 0))
_degs = _part
_out = jax.ShapeDtypeStruct((_N, _D), jnp.float32)


def _k1(x, W1, degs):
    return pl.pallas_call(
        _k1_body, grid=(_N // _R,),
        in_specs=[_row, _full, _degs],
        out_specs=_row, out_shape=_out,
    )(x, W1, degs)


def _k2(p, hs1, degs, b1, W2):
    return pl.pallas_call(
        _k2_body, grid=(_N // _R,),
        in_specs=[_part, _row, _degs, _bias, _full],
        out_specs=_row, out_shape=_out,
    )(p, hs1, degs, b1, W2)


def _k3(q, hs2, degs, b2, Wu, bu, Ws, bs):
    return pl.pallas_call(
        _k3_body, grid=(_N // _R,),
        in_specs=[_part, _row, _degs, _bias, _full, _bias, _full, _bias],
        out_specs=(_row, _row), out_shape=(_out, _out),
    )(q, hs2, degs, b2, Wu, bu, Ws, bs)


def kernel(initial_e_emb, edge_index, W1, b1, W2, b2, Wu, bu, Ws, bs):
    ei = edge_index.astype(jnp.int32)
    npad = _EP - _E
    pad = jnp.zeros((npad,), jnp.int32)
    src = jnp.concatenate([ei[0], pad]).reshape(_EP // _CHUNK, _CHUNK)
    dst = jnp.concatenate([ei[1], pad]).reshape(_EP // _CHUNK, _CHUNK)
    zeros = jnp.zeros((_N, _D), jnp.float32)
    ones = jnp.ones((_CHUNK, _D), jnp.float32)

    degs = _sc_degree(dst, ones, zeros)
    hs1 = _k1(initial_e_emb, W1, degs)
    p = _sc_agg(hs1, src, dst, zeros)
    hs2 = _k2(p, hs1, degs, b1.reshape(1, _D), W2)
    q = _sc_agg(hs2, src, dst, zeros)
    u, sigma = _k3(q, hs2, degs, b2.reshape(1, _D),
                   Wu, bu.reshape(1, _D), Ws, bs.reshape(1, _D))
    return (u, sigma)


# speedup vs baseline: 28.5596x; 1.0665x over previous
import functools

import jax
import jax.numpy as jnp
from jax import lax
from jax.experimental import pallas as pl
from jax.experimental.pallas import tpu as pltpu
from jax.experimental.pallas import tpu_sc as plsc

_N = 10000
_D = 128
_E = 320000
_NC = 2
_NS = 16
_NW = _NC * _NS
_CHUNK = 128
_WCH = 80
_NCH = _E // _CHUNK
_EP = _NW * _WCH * _CHUNK
_DW = 64
_RPS = 624
_TAIL0 = _RPS * _NS
_TAILN = _N - _TAIL0
_R = 1000

_mesh = plsc.VectorSubcoreMesh(core_axis_name="c", subcore_axis_name="s")


def _zero_acc(z_hbm, acc, sid):
    r0 = sid * _RPS
    pltpu.sync_copy(z_hbm.at[pl.ds(r0, _RPS)], acc.at[pl.ds(r0, _RPS)])

    @pl.when(sid == _NS - 1)
    def _():
        pltpu.sync_copy(z_hbm.at[pl.ds(_TAIL0, _TAILN)],
                        acc.at[pl.ds(_TAIL0, _TAILN)])


def _writeback(acc, out_hbm, cid, sid):
    r0 = sid * _RPS
    pltpu.sync_copy(acc.at[pl.ds(r0, _RPS)], out_hbm.at[cid, pl.ds(r0, _RPS)])

    @pl.when(sid == _NS - 1)
    def _():
        pltpu.sync_copy(acc.at[pl.ds(_TAIL0, _TAILN)],
                        out_hbm.at[cid, pl.ds(_TAIL0, _TAILN)])


@functools.partial(
    pl.kernel,
    out_type=jax.ShapeDtypeStruct((_NC, _N, _DW), jnp.float32),
    mesh=_mesh,
    scratch_types=[
        pltpu.VMEM((_WCH, _CHUNK), jnp.int32),
        pltpu.VMEM((_CHUNK, _DW), jnp.float32),
        pltpu.VMEM_SHARED((_N, _DW), jnp.float32),
    ],
)
def _sc_degree(dst_hbm, ones_hbm, z_hbm, out_hbm, dbuf, ones_v, acc):
    cid = lax.axis_index("c")
    sid = lax.axis_index("s")
    wid = cid * _NS + sid
    nch = jnp.minimum(_WCH, _NCH - wid * _WCH)
    pltpu.sync_copy(dst_hbm.at[pl.ds(wid * _WCH, _WCH)], dbuf)
    _zero_acc(z_hbm, acc, sid)
    pltpu.sync_copy(ones_hbm, ones_v)
    plsc.subcore_barrier()

    @pl.loop(0, _WCH)
    def _(j):
        @pl.when(j < nch)
        def _():
            pltpu.sync_copy(ones_v, acc.at[dbuf.at[j]], add=True)

    plsc.subcore_barrier()
    _writeback(acc, out_hbm, cid, sid)


@functools.partial(
    pl.kernel,
    out_type=jax.ShapeDtypeStruct((_NC, _N, _D), jnp.float32),
    mesh=_mesh,
    scratch_types=[
        pltpu.VMEM((_WCH // 2, _CHUNK), jnp.int32),
        pltpu.VMEM((_WCH // 2, _CHUNK), jnp.int32),
        pltpu.VMEM((2, _CHUNK, _D), jnp.float32),
        pltpu.VMEM_SHARED((_N, _D), jnp.float32),
        pltpu.SemaphoreType.DMA,
        pltpu.SemaphoreType.DMA,
    ],
)
def _sc_agg(hs_hbm, src_hbm, dst_hbm, z_hbm, out_hbm, sbuf, dbuf, rbuf, acc,
            sem0, sem1):
    cid = lax.axis_index("c")
    sid = lax.axis_index("s")
    wid = cid * _NS + sid
    nch = jnp.minimum(_WCH, _NCH - wid * _WCH)
    _zero_acc(z_hbm, acc, sid)
    plsc.subcore_barrier()

    hw = _WCH // 2
    for p in range(2):
        cnt = jnp.clip(nch - p * hw, 0, hw)

        @pl.when(cnt > 0)
        def _():
            base = wid * _WCH + p * hw
            pltpu.sync_copy(src_hbm.at[pl.ds(base, hw)], sbuf)
            pltpu.sync_copy(dst_hbm.at[pl.ds(base, hw)], dbuf)
            pltpu.async_copy(hs_hbm.at[sbuf.at[0]], rbuf.at[0], sem0)

            @pl.loop(0, hw // 2)
            def _(k):
                @pl.when(k < cnt // 2)
                def _():
                    j = 2 * k
                    c1 = pltpu.async_copy(hs_hbm.at[sbuf.at[j + 1]],
                                          rbuf.at[1], sem1)
                    pltpu.make_async_copy(hs_hbm.at[sbuf.at[j]], rbuf.at[0],
                                          sem0).wait()
                    pltpu.sync_copy(rbuf.at[0], acc.at[dbuf.at[j]], add=True)

                    @pl.when(j + 2 < cnt)
                    def _():
                        pltpu.async_copy(hs_hbm.at[sbuf.at[j + 2]],
                                         rbuf.at[0], sem0)

                    c1.wait()
                    pltpu.sync_copy(rbuf.at[1], acc.at[dbuf.at[j + 1]],
                                    add=True)

    plsc.subcore_barrier()
    _writeback(acc, out_hbm, cid, sid)


def _dis_from(d_ref):
    deg = 1.0 + d_ref[0, :, 0:1] + d_ref[1, :, 0:1]
    return lax.rsqrt(deg)


def _k1_body(x_ref, w_ref, d_ref, o_ref):
    dis = _dis_from(d_ref)
    o_ref[...] = dis * jnp.dot(x_ref[...], w_ref[...],
                               preferred_element_type=jnp.float32)


def _k2_body(p_ref, hs_ref, d_ref, b1_ref, w2_ref, o_ref):
    dis = _dis_from(d_ref)
    s = p_ref[0] + p_ref[1] + hs_ref[...]
    h = jnp.maximum(dis * s + b1_ref[...], 0.0)
    o_ref[...] = dis * jnp.dot(h, w2_ref[...],
                               preferred_element_type=jnp.float32)


def _k3_body(q_ref, hs_ref, d_ref, b2_ref, wu_ref, bu_ref, ws_ref, bs_ref,
             u_ref, s_ref):
    dis = _dis_from(d_ref)
    hh = dis * (q_ref[0] + q_ref[1] + hs_ref[...]) + b2_ref[...]
    u_ref[...] = jnp.dot(hh, wu_ref[...],
                         preferred_element_type=jnp.float32) + bu_ref[...]
    sg = jnp.dot(hh, ws_ref[...],
                 preferred_element_type=jnp.float32) + bs_ref[...]
    s_ref[...] = jnp.maximum(sg, 0.01)


_row = pl.BlockSpec((_R, _D), lambda i: (i, 0))
_full = pl.BlockSpec((_D, _D), lambda i: (0, 0))
_bias = pl.BlockSpec((1, _D), lambda i: (0, 0))
_part = pl.BlockSpec((_NC, _R, _D), lambda i: (0, i, 0))
_degs = pl.BlockSpec((_NC, _R, _DW), lambda i: (0, i, 0))
_out = jax.ShapeDtypeStruct((_N, _D), jnp.float32)


def _k1(x, W1, degs):
    return pl.pallas_call(
        _k1_body, grid=(_N // _R,),
        in_specs=[_row, _full, _degs],
        out_specs=_row, out_shape=_out,
    )(x, W1, degs)


def _k2(p, hs1, degs, b1, W2):
    return pl.pallas_call(
        _k2_body, grid=(_N // _R,),
        in_specs=[_part, _row, _degs, _bias, _full],
        out_specs=_row, out_shape=_out,
    )(p, hs1, degs, b1, W2)


def _k3(q, hs2, degs, b2, Wu, bu, Ws, bs):
    return pl.pallas_call(
        _k3_body, grid=(_N // _R,),
        in_specs=[_part, _row, _degs, _bias, _full, _bias, _full, _bias],
        out_specs=(_row, _row), out_shape=(_out, _out),
    )(q, hs2, degs, b2, Wu, bu, Ws, bs)


def kernel(initial_e_emb, edge_index, W1, b1, W2, b2, Wu, bu, Ws, bs):
    ei = edge_index.astype(jnp.int32)
    npad = _EP - _E
    pad = jnp.zeros((npad,), jnp.int32)
    src = jnp.concatenate([ei[0], pad]).reshape(_EP // _CHUNK, _CHUNK)
    dst = jnp.concatenate([ei[1], pad]).reshape(_EP // _CHUNK, _CHUNK)
    zeros = jnp.zeros((_N, _D), jnp.float32)
    zeros_dw = jnp.zeros((_N, _DW), jnp.float32)
    ones = jnp.ones((_CHUNK, _DW), jnp.float32)

    degs = _sc_degree(dst, ones, zeros_dw)
    hs1 = _k1(initial_e_emb, W1, degs)
    p = _sc_agg(hs1, src, dst, zeros)
    hs2 = _k2(p, hs1, degs, b1.reshape(1, _D), W2)
    q = _sc_agg(hs2, src, dst, zeros)
    u, sigma = _k3(q, hs2, degs, b2.reshape(1, _D),
                   Wu, bu.reshape(1, _D), Ws, bs.reshape(1, _D))
    return (u, sigma)


# speedup vs baseline: 30.3475x; 1.0626x over previous
import functools

import jax
import jax.numpy as jnp
from jax import lax
from jax.experimental import pallas as pl
from jax.experimental.pallas import tpu as pltpu
from jax.experimental.pallas import tpu_sc as plsc

_N = 10000
_D = 128
_E = 320000
_NC = 2
_NS = 16
_NW = _NC * _NS
_CHUNK = 128
_WCH = 80
_NCH = _E // _CHUNK
_EP = _NW * _WCH * _CHUNK
_DW = 64
_RPS = 624
_TAIL0 = _RPS * _NS
_TAILN = _N - _TAIL0
_R = 1000

_mesh = plsc.VectorSubcoreMesh(core_axis_name="c", subcore_axis_name="s")


def _fill(buf, width, value):
    @pl.loop(0, _CHUNK)
    def _(r):
        for c in range(width // 16):
            buf[r, pl.ds(c * 16, 16)] = jnp.full((16,), value, jnp.float32)


def _zero_acc(zs, acc, sid):
    r0 = sid * _RPS
    for t in range(4):
        pltpu.sync_copy(zs(_CHUNK), acc.at[pl.ds(r0 + t * _CHUNK, _CHUNK)])
    pltpu.sync_copy(zs(_RPS - 4 * _CHUNK),
                    acc.at[pl.ds(r0 + 4 * _CHUNK, _RPS - 4 * _CHUNK)])

    @pl.when(sid == _NS - 1)
    def _():
        pltpu.sync_copy(zs(_TAILN), acc.at[pl.ds(_TAIL0, _TAILN)])


def _writeback(acc, out_hbm, cid, sid):
    r0 = sid * _RPS
    pltpu.sync_copy(acc.at[pl.ds(r0, _RPS)], out_hbm.at[cid, pl.ds(r0, _RPS)])

    @pl.when(sid == _NS - 1)
    def _():
        pltpu.sync_copy(acc.at[pl.ds(_TAIL0, _TAILN)],
                        out_hbm.at[cid, pl.ds(_TAIL0, _TAILN)])


@functools.partial(
    pl.kernel,
    out_type=jax.ShapeDtypeStruct((_NC, _N, _DW), jnp.float32),
    mesh=_mesh,
    scratch_types=[
        pltpu.VMEM((_WCH, _CHUNK), jnp.int32),
        pltpu.VMEM((_CHUNK, _DW), jnp.float32),
        pltpu.VMEM_SHARED((_N, _DW), jnp.float32),
        pltpu.SemaphoreType.DMA,
    ],
)
def _sc_degree(dst_hbm, out_hbm, dbuf, ones_v, acc, sem):
    cid = lax.axis_index("c")
    sid = lax.axis_index("s")
    wid = cid * _NS + sid
    nch = jnp.minimum(_WCH, _NCH - wid * _WCH)
    cd = pltpu.async_copy(dst_hbm.at[pl.ds(wid * _WCH, _WCH)], dbuf, sem)
    _fill(ones_v, _DW, 0.0)
    _zero_acc(lambda n: ones_v.at[pl.ds(0, n)], acc, sid)
    _fill(ones_v, _DW, 1.0)
    cd.wait()
    plsc.subcore_barrier()

    @pl.loop(0, _WCH)
    def _(j):
        @pl.when(j < nch)
        def _():
            pltpu.sync_copy(ones_v, acc.at[dbuf.at[j]], add=True)

    plsc.subcore_barrier()
    _writeback(acc, out_hbm, cid, sid)


@functools.partial(
    pl.kernel,
    out_type=jax.ShapeDtypeStruct((_NC, _N, _D), jnp.float32),
    mesh=_mesh,
    scratch_types=[
        pltpu.VMEM((_WCH // 2, _CHUNK), jnp.int32),
        pltpu.VMEM((_WCH // 2, _CHUNK), jnp.int32),
        pltpu.VMEM((2, _CHUNK, _D), jnp.float32),
        pltpu.VMEM_SHARED((_N, _D), jnp.float32),
        pltpu.SemaphoreType.DMA,
        pltpu.SemaphoreType.DMA,
    ],
)
def _sc_agg(hs_hbm, src_hbm, dst_hbm, out_hbm, sbuf, dbuf, rbuf, acc,
            sem0, sem1):
    cid = lax.axis_index("c")
    sid = lax.axis_index("s")
    wid = cid * _NS + sid
    nch = jnp.minimum(_WCH, _NCH - wid * _WCH)
    hw = _WCH // 2
    cs0 = pltpu.async_copy(src_hbm.at[pl.ds(wid * _WCH, hw)], sbuf, sem0)
    cd0 = pltpu.async_copy(dst_hbm.at[pl.ds(wid * _WCH, hw)], dbuf, sem1)

    @pl.loop(0, _CHUNK)
    def _(r):
        for c in range(_D // 16):
            rbuf[0, r, pl.ds(c * 16, 16)] = jnp.zeros((16,), jnp.float32)

    _zero_acc(lambda n: rbuf.at[0, pl.ds(0, n)], acc, sid)
    cs0.wait()
    cd0.wait()
    plsc.subcore_barrier()

    for p in range(2):
        cnt = jnp.clip(nch - p * hw, 0, hw)

        @pl.when(cnt > 0)
        def _():
            if p == 1:
                base = wid * _WCH + hw
                pltpu.sync_copy(src_hbm.at[pl.ds(base, hw)], sbuf)
                pltpu.sync_copy(dst_hbm.at[pl.ds(base, hw)], dbuf)
            pltpu.async_copy(hs_hbm.at[sbuf.at[0]], rbuf.at[0], sem0)

            @pl.loop(0, hw // 2)
            def _(k):
                @pl.when(k < cnt // 2)
                def _():
                    j = 2 * k
                    c1 = pltpu.async_copy(hs_hbm.at[sbuf.at[j + 1]],
                                          rbuf.at[1], sem1)
                    pltpu.make_async_copy(hs_hbm.at[sbuf.at[j]], rbuf.at[0],
                                          sem0).wait()
                    pltpu.sync_copy(rbuf.at[0], acc.at[dbuf.at[j]], add=True)

                    @pl.when(j + 2 < cnt)
                    def _():
                        pltpu.async_copy(hs_hbm.at[sbuf.at[j + 2]],
                                         rbuf.at[0], sem0)

                    c1.wait()
                    pltpu.sync_copy(rbuf.at[1], acc.at[dbuf.at[j + 1]],
                                    add=True)

    plsc.subcore_barrier()
    _writeback(acc, out_hbm, cid, sid)


def _dis_from(d_ref):
    deg = 1.0 + d_ref[0, :, 0:1] + d_ref[1, :, 0:1]
    return lax.rsqrt(deg)


def _k1_body(x_ref, w_ref, d_ref, o_ref):
    dis = _dis_from(d_ref)
    o_ref[...] = dis * jnp.dot(x_ref[...], w_ref[...],
                               preferred_element_type=jnp.float32)


def _k2_body(p_ref, hs_ref, d_ref, b1_ref, w2_ref, o_ref):
    dis = _dis_from(d_ref)
    s = p_ref[0] + p_ref[1] + hs_ref[...]
    h = jnp.maximum(dis * s + b1_ref[...], 0.0)
    o_ref[...] = dis * jnp.dot(h, w2_ref[...],
                               preferred_element_type=jnp.float32)


def _k3_body(q_ref, hs_ref, d_ref, b2_ref, wu_ref, bu_ref, ws_ref, bs_ref,
             u_ref, s_ref):
    dis = _dis_from(d_ref)
    hh = dis * (q_ref[0] + q_ref[1] + hs_ref[...]) + b2_ref[...]
    u_ref[...] = jnp.dot(hh, wu_ref[...],
                         preferred_element_type=jnp.float32) + bu_ref[...]
    sg = jnp.dot(hh, ws_ref[...],
                 preferred_element_type=jnp.float32) + bs_ref[...]
    s_ref[...] = jnp.maximum(sg, 0.01)


_row = pl.BlockSpec((_R, _D), lambda i: (i, 0))
_full = pl.BlockSpec((_D, _D), lambda i: (0, 0))
_bias = pl.BlockSpec((1, _D), lambda i: (0, 0))
_part = pl.BlockSpec((_NC, _R, _D), lambda i: (0, i, 0))
_degs = pl.BlockSpec((_NC, _R, _DW), lambda i: (0, i, 0))
_out = jax.ShapeDtypeStruct((_N, _D), jnp.float32)


def _k1(x, W1, degs):
    return pl.pallas_call(
        _k1_body, grid=(_N // _R,),
        in_specs=[_row, _full, _degs],
        out_specs=_row, out_shape=_out,
    )(x, W1, degs)


def _k2(p, hs1, degs, b1, W2):
    return pl.pallas_call(
        _k2_body, grid=(_N // _R,),
        in_specs=[_part, _row, _degs, _bias, _full],
        out_specs=_row, out_shape=_out,
    )(p, hs1, degs, b1, W2)


def _k3(q, hs2, degs, b2, Wu, bu, Ws, bs):
    return pl.pallas_call(
        _k3_body, grid=(_N // _R,),
        in_specs=[_part, _row, _degs, _bias, _full, _bias, _full, _bias],
        out_specs=(_row, _row), out_shape=(_out, _out),
    )(q, hs2, degs, b2, Wu, bu, Ws, bs)


def kernel(initial_e_emb, edge_index, W1, b1, W2, b2, Wu, bu, Ws, bs):
    ei = edge_index.astype(jnp.int32)
    npad = _EP - _E
    pad = jnp.zeros((npad,), jnp.int32)
    src = jnp.concatenate([ei[0], pad]).reshape(_EP // _CHUNK, _CHUNK)
    dst = jnp.concatenate([ei[1], pad]).reshape(_EP // _CHUNK, _CHUNK)
    degs = _sc_degree(dst)
    hs1 = _k1(initial_e_emb, W1, degs)
    p = _sc_agg(hs1, src, dst)
    hs2 = _k2(p, hs1, degs, b1.reshape(1, _D), W2)
    q = _sc_agg(hs2, src, dst)
    u, sigma = _k3(q, hs2, degs, b2.reshape(1, _D),
                   Wu, bu.reshape(1, _D), Ws, bs.reshape(1, _D))
    return (u, sigma)
